# baseline (speedup 1.0000x reference)
import functools

import jax
import jax.numpy as jnp
from jax import lax
from jax.experimental import pallas as pl
from jax.experimental.pallas import tpu as pltpu
from jax.experimental.pallas import tpu_sc as plsc

EMBED_DIM = 64
NUM_WORKERS = 32
ROWS_PER_STREAM = 128
STREAMS_PER_CHUNK = 4
CHUNK = ROWS_PER_STREAM * STREAMS_PER_CHUNK


def _sc_gather(idx2d, table, num_idx):
  per_w = num_idx // NUM_WORKERS
  n_chunks = per_w // CHUNK
  idx_rows_per_w = per_w // ROWS_PER_STREAM

  mesh = plsc.VectorSubcoreMesh(core_axis_name="c", subcore_axis_name="s")

  @functools.partial(
      pl.kernel,
      mesh=mesh,
      compiler_params=pltpu.CompilerParams(use_tc_tiling_on_sc=False),
      out_type=jax.ShapeDtypeStruct((num_idx, EMBED_DIM), jnp.float32),
      scratch_types=[
          pltpu.VMEM((STREAMS_PER_CHUNK, ROWS_PER_STREAM), jnp.int32),
          pltpu.VMEM((CHUNK, EMBED_DIM), jnp.float32),
          pltpu.SemaphoreType.DMA,
      ],
  )
  def k(idx_hbm, table_hbm, out_hbm, idx_v, rows_v, sem):
    wid = lax.axis_index("s") * 2 + lax.axis_index("c")
    idx_row_base = wid * idx_rows_per_w
    out_base = wid * per_w

    def body(c, carry):
      r0 = idx_row_base + c * STREAMS_PER_CHUNK
      pltpu.sync_copy(idx_hbm.at[pl.ds(r0, STREAMS_PER_CHUNK)], idx_v)
      copies = []
      for j in range(STREAMS_PER_CHUNK):
        copies.append(pltpu.async_copy(
            table_hbm.at[idx_v.at[j]],
            rows_v.at[pl.ds(j * ROWS_PER_STREAM, ROWS_PER_STREAM)],
            sem))
      for cp in copies:
        cp.wait()
      pltpu.sync_copy(rows_v, out_hbm.at[pl.ds(out_base + c * CHUNK, CHUNK)])
      return carry

    lax.fori_loop(0, n_chunks, body, 0)

  return k(idx2d, table)


def kernel(indices, glove_vectors):
  b, h = indices.shape
  num_idx = b * h
  idx2d = indices.reshape(num_idx // ROWS_PER_STREAM,
                          ROWS_PER_STREAM).astype(jnp.int32)
  out = _sc_gather(idx2d, glove_vectors, num_idx)
  return out.reshape(b, h, EMBED_DIM)


# speedup vs baseline: 1.0355x; 1.0355x over previous
import functools

import jax
import jax.numpy as jnp
from jax import lax
from jax.experimental import pallas as pl
from jax.experimental.pallas import tpu as pltpu
from jax.experimental.pallas import tpu_sc as plsc

EMBED_DIM = 64
NUM_WORKERS = 32
ROWS_PER_STREAM = 128
STREAMS_PER_CHUNK = 4
CHUNK = ROWS_PER_STREAM * STREAMS_PER_CHUNK


def _sc_gather(idx2d, table, num_idx):
  per_w = num_idx // NUM_WORKERS
  n_chunks = per_w // CHUNK
  n_pairs = n_chunks // 2
  idx_rows_per_w = per_w // ROWS_PER_STREAM

  mesh = plsc.VectorSubcoreMesh(core_axis_name="c", subcore_axis_name="s")

  @functools.partial(
      pl.kernel,
      mesh=mesh,
      compiler_params=pltpu.CompilerParams(use_tc_tiling_on_sc=False),
      out_type=jax.ShapeDtypeStruct((num_idx, EMBED_DIM), jnp.float32),
      scratch_types=[
          pltpu.VMEM((STREAMS_PER_CHUNK, ROWS_PER_STREAM), jnp.int32),
          pltpu.VMEM((STREAMS_PER_CHUNK, ROWS_PER_STREAM), jnp.int32),
          pltpu.VMEM((CHUNK, EMBED_DIM), jnp.float32),
          pltpu.VMEM((CHUNK, EMBED_DIM), jnp.float32),
          pltpu.SemaphoreType.DMA,
          pltpu.SemaphoreType.DMA,
      ],
  )
  def k(idx_hbm, table_hbm, out_hbm, idx0, idx1, rows0, rows1, sem0, sem1):
    wid = lax.axis_index("s") * 2 + lax.axis_index("c")
    idx_row_base = wid * idx_rows_per_w
    out_base = wid * per_w

    def gather_chunk(idx_v, rows_v, sem, g):
      r0 = idx_row_base + g * STREAMS_PER_CHUNK
      pltpu.sync_copy(idx_hbm.at[pl.ds(r0, STREAMS_PER_CHUNK)], idx_v)
      for j in range(STREAMS_PER_CHUNK):
        pltpu.async_copy(table_hbm.at[idx_v.at[j]],
                         rows_v.at[pl.ds(j * ROWS_PER_STREAM,
                                         ROWS_PER_STREAM)], sem)

    def wait_and_store(rows_v, sem, g):
      pltpu.make_async_copy(out_hbm.at[pl.ds(0, CHUNK)], rows_v, sem).wait()
      pltpu.sync_copy(rows_v, out_hbm.at[pl.ds(out_base + g * CHUNK, CHUNK)])

    def body(p, carry):
      ga = 2 * p

      gather_chunk(idx0, rows0, sem0, ga)

      @pl.when(p > 0)
      def _():
        wait_and_store(rows1, sem1, ga - 1)

      gather_chunk(idx1, rows1, sem1, ga + 1)
      wait_and_store(rows0, sem0, ga)
      return carry

    lax.fori_loop(0, n_pairs, body, 0)
    wait_and_store(rows1, sem1, n_chunks - 1)

  return k(idx2d, table)


def kernel(indices, glove_vectors):
  b, h = indices.shape
  num_idx = b * h
  idx2d = indices.reshape(num_idx // ROWS_PER_STREAM,
                          ROWS_PER_STREAM).astype(jnp.int32)
  out = _sc_gather(idx2d, glove_vectors, num_idx)
  return out.reshape(b, h, EMBED_DIM)


# speedup vs baseline: 1.0364x; 1.0008x over previous
import functools

import jax
import jax.numpy as jnp
from jax import lax
from jax.experimental import pallas as pl
from jax.experimental.pallas import tpu as pltpu
from jax.experimental.pallas import tpu_sc as plsc

EMBED_DIM = 64
NUM_WORKERS = 32
ROWS_PER_STREAM = 512
STREAMS_PER_CHUNK = 1
CHUNK = ROWS_PER_STREAM * STREAMS_PER_CHUNK


def _sc_gather(idx2d, table, num_idx):
  per_w = num_idx // NUM_WORKERS
  n_chunks = per_w // CHUNK
  n_pairs = n_chunks // 2
  idx_rows_per_w = per_w // ROWS_PER_STREAM

  mesh = plsc.VectorSubcoreMesh(core_axis_name="c", subcore_axis_name="s")

  @functools.partial(
      pl.kernel,
      mesh=mesh,
      compiler_params=pltpu.CompilerParams(use_tc_tiling_on_sc=False),
      out_type=jax.ShapeDtypeStruct((num_idx, EMBED_DIM), jnp.float32),
      scratch_types=[
          pltpu.VMEM((STREAMS_PER_CHUNK, ROWS_PER_STREAM), jnp.int32),
          pltpu.VMEM((STREAMS_PER_CHUNK, ROWS_PER_STREAM), jnp.int32),
          pltpu.VMEM((CHUNK, EMBED_DIM), jnp.float32),
          pltpu.VMEM((CHUNK, EMBED_DIM), jnp.float32),
          pltpu.SemaphoreType.DMA,
          pltpu.SemaphoreType.DMA,
      ],
  )
  def k(idx_hbm, table_hbm, out_hbm, idx0, idx1, rows0, rows1, sem0, sem1):
    wid = lax.axis_index("s") * 2 + lax.axis_index("c")
    idx_row_base = wid * idx_rows_per_w
    out_base = wid * per_w

    def gather_chunk(idx_v, rows_v, sem, g):
      r0 = idx_row_base + g * STREAMS_PER_CHUNK
      pltpu.sync_copy(idx_hbm.at[pl.ds(r0, STREAMS_PER_CHUNK)], idx_v)
      for j in range(STREAMS_PER_CHUNK):
        pltpu.async_copy(table_hbm.at[idx_v.at[j]],
                         rows_v.at[pl.ds(j * ROWS_PER_STREAM,
                                         ROWS_PER_STREAM)], sem)

    def wait_and_store(rows_v, sem, g):
      pltpu.make_async_copy(out_hbm.at[pl.ds(0, CHUNK)], rows_v, sem).wait()
      pltpu.sync_copy(rows_v, out_hbm.at[pl.ds(out_base + g * CHUNK, CHUNK)])

    def body(p, carry):
      ga = 2 * p

      gather_chunk(idx0, rows0, sem0, ga)

      @pl.when(p > 0)
      def _():
        wait_and_store(rows1, sem1, ga - 1)

      gather_chunk(idx1, rows1, sem1, ga + 1)
      wait_and_store(rows0, sem0, ga)
      return carry

    lax.fori_loop(0, n_pairs, body, 0)
    wait_and_store(rows1, sem1, n_chunks - 1)

  return k(idx2d, table)


def kernel(indices, glove_vectors):
  b, h = indices.shape
  num_idx = b * h
  idx2d = indices.reshape(num_idx // ROWS_PER_STREAM,
                          ROWS_PER_STREAM).astype(jnp.int32)
  out = _sc_gather(idx2d, glove_vectors, num_idx)
  return out.reshape(b, h, EMBED_DIM)
